# baseline (speedup 1.0000x reference)
import jax
import jax.numpy as jnp
from jax.experimental import pallas as pl
from jax.experimental.pallas import tpu as pltpu

_TRACKS_X = 12
_CLUSTERS_X = 8
_G = 256
_N = 200000
_B = 4000
_NB = _N // _B
_WIN = 16
_W = 128
_E = 34
_DOUT = 256


def _elu(x):
    return jnp.maximum(x, 0.0) + jnp.exp(jnp.minimum(x, 0.0)) - 1.0


def _body(span_ref,
          xT_ref, t_ref, b_ref,
          w1tT_ref, b1t_ref, w1cT_ref, b1c_ref,
          we2tT_ref, be2t_ref, we2cT_ref, be2c_ref,
          wd1T_ref, bd1_ref, wd2_ref, bd2_ref,
          out_t_ref, out_c_ref,
          wmt_ref, bmt_ref, wmc_ref, bmc_ref,
          st_ref, sc_ref, ct_ref, cc_ref):
    i = pl.program_id(0)

    @pl.when(i == 0)
    def _init():
        wd1T = wd1T_ref[...]
        wmt_ref[...] = jnp.dot(wd1T, we2tT_ref[...], preferred_element_type=jnp.float32)
        wmc_ref[...] = jnp.dot(wd1T, we2cT_ref[...], preferred_element_type=jnp.float32)
        bmt_ref[...] = jnp.dot(wd1T, be2t_ref[...], preferred_element_type=jnp.float32) + bd1_ref[...]
        bmc_ref[...] = jnp.dot(wd1T, be2c_ref[...], preferred_element_type=jnp.float32) + bd1_ref[...]
        st_ref[...] = jnp.zeros_like(st_ref)
        sc_ref[...] = jnp.zeros_like(sc_ref)
        ct_ref[...] = jnp.zeros_like(ct_ref)
        cc_ref[...] = jnp.zeros_like(cc_ref)

    xT = xT_ref[...].T
    trow = t_ref[0]
    brow = b_ref[0]
    is_t = trow == 1

    z1t = jnp.dot(w1tT_ref[...], xT, preferred_element_type=jnp.float32)
    z1c = jnp.dot(w1cT_ref[...], xT, preferred_element_type=jnp.float32)
    z1 = jnp.where(is_t, z1t, z1c) + jnp.where(is_t, b1t_ref[...], b1c_ref[...])
    h1 = _elu(z1)
    wt = is_t.astype(jnp.float32)
    h1t_m = h1 * wt
    z2 = (jnp.dot(wmt_ref[...], h1t_m, preferred_element_type=jnp.float32)
          + jnp.dot(wmc_ref[...], h1 - h1t_m, preferred_element_type=jnp.float32)
          + jnp.where(is_t, bmt_ref[...], bmc_ref[...]))
    h = _elu(z2)

    gfirst = span_ref[i, 0]
    glast = span_ref[i, 1]
    base = jnp.minimum((gfirst // 8) * 8, _G - _WIN)
    dn = (((1,), (1,)), ((), ()))
    bt = jnp.where(is_t, brow, _G)
    bc = jnp.where(trow == 2, brow, _G)

    @pl.when(glast - base < _WIN)
    def _pool_windowed():
        r = jax.lax.broadcasted_iota(jnp.int32, (_WIN, _B), 0) + base
        ot = jnp.where(r == bt, 1.0, 0.0)
        oc = jnp.where(r == bc, 1.0, 0.0)
        st_ref[pl.ds(base, _WIN), :] += jax.lax.dot_general(
            ot, h, dn, preferred_element_type=jnp.float32)
        sc_ref[pl.ds(base, _WIN), :] += jax.lax.dot_general(
            oc, h, dn, preferred_element_type=jnp.float32)
        ct_ref[pl.ds(base, _WIN), :] += jnp.sum(ot, axis=1, keepdims=True)
        cc_ref[pl.ds(base, _WIN), :] += jnp.sum(oc, axis=1, keepdims=True)

    @pl.when(glast - base >= _WIN)
    def _pool_full():
        r = jax.lax.broadcasted_iota(jnp.int32, (_G, _B), 0)
        ot = jnp.where(r == bt, 1.0, 0.0)
        oc = jnp.where(r == bc, 1.0, 0.0)
        st_ref[...] += jax.lax.dot_general(ot, h, dn, preferred_element_type=jnp.float32)
        sc_ref[...] += jax.lax.dot_general(oc, h, dn, preferred_element_type=jnp.float32)
        ct_ref[...] += jnp.sum(ot, axis=1, keepdims=True)
        cc_ref[...] += jnp.sum(oc, axis=1, keepdims=True)

    @pl.when(i == _NB - 1)
    def _fin():
        wd2 = wd2_ref[...]
        bd2 = bd2_ref[...]
        cnt_t = ct_ref[...]
        mean_t = st_ref[...] / jnp.maximum(cnt_t, 1.0)
        pt = jnp.dot(mean_t, wd2, preferred_element_type=jnp.float32) + bd2
        out_t_ref[...] = jnp.where(cnt_t > 0, pt, 0.0)
        cnt_c = cc_ref[...]
        mean_c = sc_ref[...] / jnp.maximum(cnt_c, 1.0)
        pc = jnp.dot(mean_c, wd2, preferred_element_type=jnp.float32) + bd2
        out_c_ref[...] = jnp.where(cnt_c > 0, pc, 0.0)


@jax.jit
def kernel(x_feat, type_id, batch_idx, We_t1, be_t1, We_t2, be_t2,
           We_c1, be_c1, We_c2, be_c2, Wd1, bd1, Wd2, bd2):
    f32 = jnp.float32
    nfeat = x_feat.shape[1]
    w1tT = jnp.zeros((_W, nfeat), f32).at[:, :_TRACKS_X].set(We_t1.T)
    w1cT = jnp.zeros((_W, nfeat), f32).at[:, :_CLUSTERS_X].set(We_c1.T)
    bi = batch_idx.astype(jnp.int32)
    trow = type_id.astype(jnp.int32).reshape(_NB, 1, _B)
    brow = bi.reshape(_NB, 1, _B)
    blk = bi.reshape(_NB, _B)
    spans = jnp.stack([blk[:, 0], blk[:, -1]], axis=1)

    const = lambda *dims: pl.BlockSpec(dims, lambda i, s: tuple(0 for _ in dims))
    grid_spec = pltpu.PrefetchScalarGridSpec(
        num_scalar_prefetch=1,
        grid=(_NB,),
        in_specs=[
            pl.BlockSpec((_B, nfeat), lambda i, s: (i, 0)),
            pl.BlockSpec((1, 1, _B), lambda i, s: (i, 0, 0)),
            pl.BlockSpec((1, 1, _B), lambda i, s: (i, 0, 0)),
            const(_W, nfeat), const(_W, 1),
            const(_W, nfeat), const(_W, 1),
            const(_E, _W), const(_E, 1),
            const(_E, _W), const(_E, 1),
            const(_W, _E), const(_W, 1),
            const(_W, _DOUT), const(1, _DOUT),
        ],
        out_specs=[const(_G, _DOUT), const(_G, _DOUT)],
        scratch_shapes=[
            pltpu.VMEM((_W, _W), f32), pltpu.VMEM((_W, 1), f32),
            pltpu.VMEM((_W, _W), f32), pltpu.VMEM((_W, 1), f32),
            pltpu.VMEM((_G, _W), f32), pltpu.VMEM((_G, _W), f32),
            pltpu.VMEM((_G, 1), f32), pltpu.VMEM((_G, 1), f32),
        ],
    )
    out_t, out_c = pl.pallas_call(
        _body,
        grid_spec=grid_spec,
        out_shape=[jax.ShapeDtypeStruct((_G, _DOUT), f32),
                   jax.ShapeDtypeStruct((_G, _DOUT), f32)],
        compiler_params=pltpu.CompilerParams(
            dimension_semantics=("arbitrary",)),
    )(spans, x_feat, trow, brow,
      w1tT, be_t1.reshape(_W, 1), w1cT, be_c1.reshape(_W, 1),
      We_t2.T, be_t2.reshape(_E, 1), We_c2.T, be_c2.reshape(_E, 1),
      Wd1.T, bd1.reshape(_W, 1), Wd2, bd2.reshape(1, _DOUT))
    return (out_t, out_c)


# speedup vs baseline: 1.0936x; 1.0936x over previous
import jax
import jax.numpy as jnp
from jax.experimental import pallas as pl
from jax.experimental.pallas import tpu as pltpu

_TRACKS_X = 12
_CLUSTERS_X = 8
_G = 256
_N = 200000
_B = 4000
_NSUB = _N // _B
_NB = _NSUB // 2
_WIN = 16
_W = 128
_E = 34
_DOUT = 256


def _elu(x):
    return jnp.maximum(x, 0.0) + jnp.exp(jnp.minimum(x, 0.0)) - 1.0


def _body(span_ref,
          x0_ref, t0_ref, b0_ref, x1_ref, t1_ref, b1_ref,
          w1t_ref, b1t_ref, w1c_ref, b1c_ref,
          we2t_ref, be2t_ref, we2c_ref, be2c_ref,
          wd1_ref, bd1_ref, wd2_ref, bd2_ref,
          out_t_ref, out_c_ref,
          wmt_ref, bmt_ref, wmc_ref, bmc_ref,
          st_ref, sc_ref, ct_ref, cc_ref):
    i = pl.program_id(0)

    @pl.when(i == 0)
    def _init():
        wd1 = wd1_ref[...]
        wmt_ref[...] = jnp.dot(we2t_ref[...], wd1, preferred_element_type=jnp.float32)
        wmc_ref[...] = jnp.dot(we2c_ref[...], wd1, preferred_element_type=jnp.float32)
        bmt_ref[...] = jnp.dot(be2t_ref[...], wd1, preferred_element_type=jnp.float32) + bd1_ref[...]
        bmc_ref[...] = jnp.dot(be2c_ref[...], wd1, preferred_element_type=jnp.float32) + bd1_ref[...]
        st_ref[...] = jnp.zeros_like(st_ref)
        sc_ref[...] = jnp.zeros_like(sc_ref)
        ct_ref[...] = jnp.zeros_like(ct_ref)
        cc_ref[...] = jnp.zeros_like(cc_ref)

    def _process(x_ref, t_ref, b_ref, sidx):
        x = x_ref[...]
        trow = t_ref[0]
        brow = b_ref[0]

        h1t = _elu(jnp.dot(x, w1t_ref[...], preferred_element_type=jnp.float32) + b1t_ref[...])
        ht = _elu(jnp.dot(h1t, wmt_ref[...], preferred_element_type=jnp.float32) + bmt_ref[...])
        h1c = _elu(jnp.dot(x, w1c_ref[...], preferred_element_type=jnp.float32) + b1c_ref[...])
        hc = _elu(jnp.dot(h1c, wmc_ref[...], preferred_element_type=jnp.float32) + bmc_ref[...])

        gfirst = span_ref[sidx, 0]
        glast = span_ref[sidx, 1]
        base = jnp.minimum((gfirst // 8) * 8, _G - _WIN)
        dn = (((1,), (0,)), ((), ()))
        bt = jnp.where(trow == 1, brow, _G)
        bc = jnp.where(trow == 2, brow, _G)

        @pl.when(glast - base < _WIN)
        def _pool_windowed():
            r = jax.lax.broadcasted_iota(jnp.int32, (_WIN, _B), 0) + base
            ot = jnp.where(r == bt, 1.0, 0.0)
            oc = jnp.where(r == bc, 1.0, 0.0)
            st_ref[pl.ds(base, _WIN), :] += jax.lax.dot_general(
                ot, ht, dn, preferred_element_type=jnp.float32)
            sc_ref[pl.ds(base, _WIN), :] += jax.lax.dot_general(
                oc, hc, dn, preferred_element_type=jnp.float32)
            ct_ref[pl.ds(base, _WIN), :] += jnp.sum(ot, axis=1, keepdims=True)
            cc_ref[pl.ds(base, _WIN), :] += jnp.sum(oc, axis=1, keepdims=True)

        @pl.when(glast - base >= _WIN)
        def _pool_full():
            r = jax.lax.broadcasted_iota(jnp.int32, (_G, _B), 0)
            ot = jnp.where(r == bt, 1.0, 0.0)
            oc = jnp.where(r == bc, 1.0, 0.0)
            st_ref[...] += jax.lax.dot_general(ot, ht, dn, preferred_element_type=jnp.float32)
            sc_ref[...] += jax.lax.dot_general(oc, hc, dn, preferred_element_type=jnp.float32)
            ct_ref[...] += jnp.sum(ot, axis=1, keepdims=True)
            cc_ref[...] += jnp.sum(oc, axis=1, keepdims=True)

    _process(x0_ref, t0_ref, b0_ref, 2 * i)
    _process(x1_ref, t1_ref, b1_ref, 2 * i + 1)

    @pl.when(i == _NB - 1)
    def _fin():
        wd2 = wd2_ref[...]
        bd2 = bd2_ref[...]
        cnt_t = ct_ref[...]
        mean_t = st_ref[...] / jnp.maximum(cnt_t, 1.0)
        pt = jnp.dot(mean_t, wd2, preferred_element_type=jnp.float32) + bd2
        out_t_ref[...] = jnp.where(cnt_t > 0, pt, 0.0)
        cnt_c = cc_ref[...]
        mean_c = sc_ref[...] / jnp.maximum(cnt_c, 1.0)
        pc = jnp.dot(mean_c, wd2, preferred_element_type=jnp.float32) + bd2
        out_c_ref[...] = jnp.where(cnt_c > 0, pc, 0.0)


@jax.jit
def kernel(x_feat, type_id, batch_idx, We_t1, be_t1, We_t2, be_t2,
           We_c1, be_c1, We_c2, be_c2, Wd1, bd1, Wd2, bd2):
    f32 = jnp.float32
    nfeat = x_feat.shape[1]
    w1t = jnp.zeros((nfeat, _W), f32).at[:_TRACKS_X].set(We_t1)
    w1c = jnp.zeros((nfeat, _W), f32).at[:_CLUSTERS_X].set(We_c1)
    bi = batch_idx.astype(jnp.int32)
    trow = type_id.astype(jnp.int32).reshape(_NSUB, 1, _B)
    brow = bi.reshape(_NSUB, 1, _B)
    blk = bi.reshape(_NSUB, _B)
    spans = jnp.stack([blk[:, 0], blk[:, -1]], axis=1)

    const = lambda *dims: pl.BlockSpec(dims, lambda i, s: tuple(0 for _ in dims))
    grid_spec = pltpu.PrefetchScalarGridSpec(
        num_scalar_prefetch=1,
        grid=(_NB,),
        in_specs=[
            pl.BlockSpec((_B, nfeat), lambda i, s: (2 * i, 0)),
            pl.BlockSpec((1, 1, _B), lambda i, s: (2 * i, 0, 0)),
            pl.BlockSpec((1, 1, _B), lambda i, s: (2 * i, 0, 0)),
            pl.BlockSpec((_B, nfeat), lambda i, s: (2 * i + 1, 0)),
            pl.BlockSpec((1, 1, _B), lambda i, s: (2 * i + 1, 0, 0)),
            pl.BlockSpec((1, 1, _B), lambda i, s: (2 * i + 1, 0, 0)),
            const(nfeat, _W), const(1, _W),
            const(nfeat, _W), const(1, _W),
            const(_W, _E), const(1, _E),
            const(_W, _E), const(1, _E),
            const(_E, _W), const(1, _W),
            const(_W, _DOUT), const(1, _DOUT),
        ],
        out_specs=[const(_G, _DOUT), const(_G, _DOUT)],
        scratch_shapes=[
            pltpu.VMEM((_W, _W), f32), pltpu.VMEM((1, _W), f32),
            pltpu.VMEM((_W, _W), f32), pltpu.VMEM((1, _W), f32),
            pltpu.VMEM((_G, _W), f32), pltpu.VMEM((_G, _W), f32),
            pltpu.VMEM((_G, 1), f32), pltpu.VMEM((_G, 1), f32),
        ],
    )
    out_t, out_c = pl.pallas_call(
        _body,
        grid_spec=grid_spec,
        out_shape=[jax.ShapeDtypeStruct((_G, _DOUT), f32),
                   jax.ShapeDtypeStruct((_G, _DOUT), f32)],
        compiler_params=pltpu.CompilerParams(
            dimension_semantics=("arbitrary",)),
    )(spans, x_feat, trow, brow, x_feat, trow, brow,
      w1t, be_t1.reshape(1, _W), w1c, be_c1.reshape(1, _W),
      We_t2, be_t2.reshape(1, _E), We_c2, be_c2.reshape(1, _E),
      Wd1, bd1.reshape(1, _W), Wd2, bd2.reshape(1, _DOUT))
    return (out_t, out_c)


# speedup vs baseline: 1.1037x; 1.0093x over previous
import jax
import jax.numpy as jnp
from jax.experimental import pallas as pl
from jax.experimental.pallas import tpu as pltpu

_TRACKS_X = 12
_CLUSTERS_X = 8
_G = 256
_N = 200000
_B = 4000
_NB = _N // _B
_WIN = 16
_W = 128
_E = 34
_DOUT = 256


def _elu(x):
    return jnp.maximum(x, 0.0) + jnp.exp(jnp.minimum(x, 0.0)) - 1.0


def _body(span_ref,
          x_ref, t_ref, b_ref,
          w1t_ref, b1t_ref, w1c_ref, b1c_ref,
          we2t_ref, be2t_ref, we2c_ref, be2c_ref,
          wd1_ref, bd1_ref, wd2_ref, bd2_ref,
          out_t_ref, out_c_ref,
          wmt_ref, bmt_ref, wmc_ref, bmc_ref,
          st_ref, sc_ref, ct_ref, cc_ref):
    i = pl.program_id(0)

    @pl.when(i == 0)
    def _init():
        wd1 = wd1_ref[...]
        wmt_ref[...] = jnp.dot(we2t_ref[...], wd1, preferred_element_type=jnp.float32)
        wmc_ref[...] = jnp.dot(we2c_ref[...], wd1, preferred_element_type=jnp.float32)
        bmt_ref[...] = jnp.dot(be2t_ref[...], wd1, preferred_element_type=jnp.float32) + bd1_ref[...]
        bmc_ref[...] = jnp.dot(be2c_ref[...], wd1, preferred_element_type=jnp.float32) + bd1_ref[...]
        st_ref[...] = jnp.zeros_like(st_ref)
        sc_ref[...] = jnp.zeros_like(sc_ref)
        ct_ref[...] = jnp.zeros_like(ct_ref)
        cc_ref[...] = jnp.zeros_like(cc_ref)

    x = x_ref[...]
    trow = t_ref[0]
    brow = b_ref[0]

    h1t = _elu(jnp.dot(x, w1t_ref[...], preferred_element_type=jnp.float32) + b1t_ref[...])
    ht = _elu(jnp.dot(h1t, wmt_ref[...], preferred_element_type=jnp.float32) + bmt_ref[...])
    h1c = _elu(jnp.dot(x, w1c_ref[...], preferred_element_type=jnp.float32) + b1c_ref[...])
    hc = _elu(jnp.dot(h1c, wmc_ref[...], preferred_element_type=jnp.float32) + bmc_ref[...])

    gfirst = span_ref[i, 0]
    glast = span_ref[i, 1]
    base = jnp.minimum((gfirst // 8) * 8, _G - _WIN)
    dn = (((1,), (0,)), ((), ()))
    bt = jnp.where(trow == 1, brow, _G)
    bc = jnp.where(trow == 2, brow, _G)

    @pl.when(glast - base < _WIN)
    def _pool_windowed():
        r = jax.lax.broadcasted_iota(jnp.int32, (_WIN, _B), 0) + base
        ot = jnp.where(r == bt, 1.0, 0.0)
        oc = jnp.where(r == bc, 1.0, 0.0)
        st_ref[pl.ds(base, _WIN), :] += jax.lax.dot_general(
            ot, ht, dn, preferred_element_type=jnp.float32)
        sc_ref[pl.ds(base, _WIN), :] += jax.lax.dot_general(
            oc, hc, dn, preferred_element_type=jnp.float32)
        ct_ref[pl.ds(base, _WIN), :] += jnp.sum(ot, axis=1, keepdims=True)
        cc_ref[pl.ds(base, _WIN), :] += jnp.sum(oc, axis=1, keepdims=True)

    @pl.when(glast - base >= _WIN)
    def _pool_full():
        r = jax.lax.broadcasted_iota(jnp.int32, (_G, _B), 0)
        ot = jnp.where(r == bt, 1.0, 0.0)
        oc = jnp.where(r == bc, 1.0, 0.0)
        st_ref[...] += jax.lax.dot_general(ot, ht, dn, preferred_element_type=jnp.float32)
        sc_ref[...] += jax.lax.dot_general(oc, hc, dn, preferred_element_type=jnp.float32)
        ct_ref[...] += jnp.sum(ot, axis=1, keepdims=True)
        cc_ref[...] += jnp.sum(oc, axis=1, keepdims=True)

    @pl.when(i == _NB - 1)
    def _fin():
        wd2 = wd2_ref[...]
        bd2 = bd2_ref[...]
        cnt_t = ct_ref[...]
        mean_t = st_ref[...] / jnp.maximum(cnt_t, 1.0)
        pt = jnp.dot(mean_t, wd2, preferred_element_type=jnp.float32) + bd2
        out_t_ref[...] = jnp.where(cnt_t > 0, pt, 0.0)
        cnt_c = cc_ref[...]
        mean_c = sc_ref[...] / jnp.maximum(cnt_c, 1.0)
        pc = jnp.dot(mean_c, wd2, preferred_element_type=jnp.float32) + bd2
        out_c_ref[...] = jnp.where(cnt_c > 0, pc, 0.0)


@jax.jit
def kernel(x_feat, type_id, batch_idx, We_t1, be_t1, We_t2, be_t2,
           We_c1, be_c1, We_c2, be_c2, Wd1, bd1, Wd2, bd2):
    f32 = jnp.float32
    nfeat = x_feat.shape[1]
    w1t = jnp.zeros((nfeat, _W), f32).at[:_TRACKS_X].set(We_t1)
    w1c = jnp.zeros((nfeat, _W), f32).at[:_CLUSTERS_X].set(We_c1)
    bi = batch_idx.astype(jnp.int32)
    trow = type_id.astype(jnp.int32).reshape(_NB, 1, _B)
    brow = bi.reshape(_NB, 1, _B)
    blk = bi.reshape(_NB, _B)
    spans = jnp.stack([blk[:, 0], blk[:, -1]], axis=1)

    const = lambda *dims: pl.BlockSpec(dims, lambda i, s: tuple(0 for _ in dims))
    grid_spec = pltpu.PrefetchScalarGridSpec(
        num_scalar_prefetch=1,
        grid=(_NB,),
        in_specs=[
            pl.BlockSpec((_B, nfeat), lambda i, s: (i, 0)),
            pl.BlockSpec((1, 1, _B), lambda i, s: (i, 0, 0)),
            pl.BlockSpec((1, 1, _B), lambda i, s: (i, 0, 0)),
            const(nfeat, _W), const(1, _W),
            const(nfeat, _W), const(1, _W),
            const(_W, _E), const(1, _E),
            const(_W, _E), const(1, _E),
            const(_E, _W), const(1, _W),
            const(_W, _DOUT), const(1, _DOUT),
        ],
        out_specs=[const(_G, _DOUT), const(_G, _DOUT)],
        scratch_shapes=[
            pltpu.VMEM((_W, _W), f32), pltpu.VMEM((1, _W), f32),
            pltpu.VMEM((_W, _W), f32), pltpu.VMEM((1, _W), f32),
            pltpu.VMEM((_G, _W), f32), pltpu.VMEM((_G, _W), f32),
            pltpu.VMEM((_G, 1), f32), pltpu.VMEM((_G, 1), f32),
        ],
    )
    out_t, out_c = pl.pallas_call(
        _body,
        grid_spec=grid_spec,
        out_shape=[jax.ShapeDtypeStruct((_G, _DOUT), f32),
                   jax.ShapeDtypeStruct((_G, _DOUT), f32)],
        compiler_params=pltpu.CompilerParams(
            dimension_semantics=("arbitrary",)),
    )(spans, x_feat, trow, brow,
      w1t, be_t1.reshape(1, _W), w1c, be_c1.reshape(1, _W),
      We_t2, be_t2.reshape(1, _E), We_c2, be_c2.reshape(1, _E),
      Wd1, bd1.reshape(1, _W), Wd2, bd2.reshape(1, _DOUT))
    return (out_t, out_c)
